# baseline (speedup 1.0000x reference)
import dataclasses

import jax
import jax.numpy as jnp
from jax import lax
from jax.experimental import pallas as pl
from jax.experimental.pallas import tpu as pltpu
from jax.experimental.pallas import tpu_sc as plsc

N = 10000
D = 128
E = 320000
EPS = 1e-5
NEG_SLOPE = 0.2

NC = 2
NS = 16
L = 16
L2 = 2 * L
NW = NC * NS
EPW = E // NW
CHUNK = 64
EPW_PAD = 10240
BLK = 8
NBLKS = 20
NPAIRB = BLK // 2
GROUPS = CHUNK // L
STRIPE = 632
N_PAD = NS * STRIPE
NDEN = 10112


def _dense_body(x_ref, mask_ref, w_ref, al_ref, ar_ref, gamma_ref, beta_ref,
                h_ref, feat_ref, el_ref, er_ref):
    x = x_ref[...]
    mean = jnp.mean(x, axis=0, keepdims=True)
    var = jnp.mean((x - mean) ** 2, axis=0, keepdims=True)
    h = (x - mean) * lax.rsqrt(var + EPS) * gamma_ref[...] + beta_ref[...]
    h = jnp.maximum(h, 0.0) * mask_ref[...]
    h_ref[...] = h
    feat = jnp.dot(h, w_ref[...], preferred_element_type=jnp.float32)
    feat_ref[...] = feat
    el_ref[...] = jnp.sum(feat * al_ref[...], axis=1, keepdims=True)
    er_ref[...] = jnp.sum(feat * ar_ref[...], axis=1, keepdims=True)


def _dense(x, dropout_mask, W, al, ar, gamma, beta):
    return pl.pallas_call(
        _dense_body,
        out_shape=[
            jax.ShapeDtypeStruct((N, D), jnp.float32),
            jax.ShapeDtypeStruct((N, D), jnp.float32),
            jax.ShapeDtypeStruct((N, 1), jnp.float32),
            jax.ShapeDtypeStruct((N, 1), jnp.float32),
        ],
    )(x, dropout_mask, W, al.reshape(1, D), ar.reshape(1, D),
      gamma.reshape(1, D), beta.reshape(1, D))


def _sc_body(feat_hbm, el_hbm, er_hbm, src_hbm, dst_hbm,
             out_hbm, den_hbm,
             el_v, er_v, src_v, dst_v, rows_a, rows_b, den_v,
             acc_sh, sem_a, sem_b, sem_sa, sem_sb):
    c = lax.axis_index("c")
    s = lax.axis_index("s")
    w = c * NS + s

    pltpu.sync_copy(el_hbm, el_v)
    pltpu.sync_copy(er_hbm, er_v)

    zv = jnp.zeros((L,), jnp.float32)

    @pl.loop(0, NDEN, step=L)
    def _zero_den(i):
        den_v[pl.ds(i, L)] = zv

    @pl.loop(0, CHUNK)
    def _zero_rows(r):
        @pl.loop(0, D, step=L)
        def _zr(k):
            rows_a[r, pl.ds(k, L)] = zv

    r0 = s * STRIPE

    @pl.loop(0, (STRIPE // CHUNK) * CHUNK, step=CHUNK)
    def _zero_acc(i):
        pltpu.sync_copy(rows_a, acc_sh.at[pl.ds(r0 + i, CHUNK)])

    _REM = STRIPE - (STRIPE // CHUNK) * CHUNK
    pltpu.sync_copy(rows_a.at[pl.ds(0, _REM)],
                    acc_sh.at[pl.ds(r0 + STRIPE - _REM, _REM)])
    plsc.subcore_barrier()

    def _gat(ci, rows_v, sem):
        pltpu.async_copy(feat_hbm.at[src_v.at[ci]], rows_v, sem)

    def _gwait(ci, rows_v, sem):
        pltpu.make_async_copy(feat_hbm.at[src_v.at[ci]], rows_v, sem).wait()

    def _swait(rows_v, sem):
        pltpu.make_async_copy(rows_v, acc_sh.at[dst_v.at[0]], sem).wait()

    def _compute(ci, rows_v, sem):
        for g in range(GROUPS):
            src16 = src_v[ci, pl.ds(g * L, L)]
            dst16 = dst_v[ci, pl.ds(g * L, L)]
            e = (plsc.load_gather(el_v, [src16]) +
                 plsc.load_gather(er_v, [dst16]))
            e = jnp.where(e >= 0.0, e, e * NEG_SLOPE)
            wv = jnp.exp(e)
            plsc.addupdate_scatter(den_v, [dst16], wv)
            for j in range(L):
                r = g * L + j
                wj = wv[j]
                for k in range(D // L):
                    sl = pl.ds(k * L, L)
                    rows_v[r, sl] = rows_v[r, sl] * wj
        pltpu.async_copy(rows_v, acc_sh.at[dst_v.at[ci]], sem, add=True)

    @pl.loop(0, NBLKS)
    def _block(b):
        @pl.when(b > 0)
        def _():
            _swait(rows_a, sem_sa)
            _swait(rows_b, sem_sb)
        pltpu.sync_copy(src_hbm.at[w].at[b], src_v)
        pltpu.sync_copy(dst_hbm.at[w].at[b], dst_v)
        _gat(0, rows_a, sem_a)
        _gat(1, rows_b, sem_b)

        @pl.loop(0, NPAIRB)
        def _pair(p):
            ca = 2 * p
            _gwait(ca, rows_a, sem_a)
            _compute(ca, rows_a, sem_sa)
            _gwait(ca + 1, rows_b, sem_b)
            _compute(ca + 1, rows_b, sem_sb)

            @pl.when(ca + 2 < BLK)
            def _():
                _swait(rows_a, sem_sa)
                _gat(ca + 2, rows_a, sem_a)
                _swait(rows_b, sem_sb)
                _gat(ca + 3, rows_b, sem_b)

    _swait(rows_a, sem_sa)
    _swait(rows_b, sem_sb)
    plsc.subcore_barrier()
    pltpu.sync_copy(acc_sh.at[pl.ds(r0, STRIPE)],
                    out_hbm.at[c, pl.ds(r0, STRIPE)])
    pltpu.sync_copy(den_v, den_hbm.at[w])


def _sc_edges(feat, el_pad, er_pad, src, dst):
    mesh = plsc.VectorSubcoreMesh(core_axis_name="c", subcore_axis_name="s",
                                  num_cores=NC)
    cp = pltpu.CompilerParams()
    if "needs_layout_passes" in pltpu.CompilerParams.__dataclass_fields__:
        cp = dataclasses.replace(cp, needs_layout_passes=False)
    kern = pl.kernel(
        _sc_body,
        out_type=[
            jax.ShapeDtypeStruct((NC, N_PAD, D), jnp.float32),
            jax.ShapeDtypeStruct((NW, NDEN), jnp.float32),
        ],
        mesh=mesh,
        compiler_params=cp,
        scratch_types=[
            pltpu.VMEM((NDEN,), jnp.float32),
            pltpu.VMEM((NDEN,), jnp.float32),
            pltpu.VMEM((BLK, CHUNK), jnp.int32),
            pltpu.VMEM((BLK, CHUNK), jnp.int32),
            pltpu.VMEM((CHUNK, D), jnp.float32),
            pltpu.VMEM((CHUNK, D), jnp.float32),
            pltpu.VMEM((NDEN,), jnp.float32),
            pltpu.VMEM_SHARED((N_PAD, D), jnp.float32),
            pltpu.SemaphoreType.DMA,
            pltpu.SemaphoreType.DMA,
            pltpu.SemaphoreType.DMA,
            pltpu.SemaphoreType.DMA,
        ],
    )
    return kern(feat, el_pad, er_pad, src, dst)


def _recip_body(den_ref, out_ref):
    den = jnp.sum(den_ref[...], axis=0, keepdims=True)
    out_ref[...] = jnp.where(den == 0.0, 0.0, 1.0 / den)


def _recip(den_all):
    return pl.pallas_call(
        _recip_body,
        out_shape=jax.ShapeDtypeStruct((1, NDEN), jnp.float32),
    )(den_all)


def _combine_body(acc_ref, recip_ref, h_ref, bias_ref, out_ref):
    num = acc_ref[0, :N, :] + acc_ref[1, :N, :]
    out_ref[...] = num * recip_ref[...] + h_ref[...] + bias_ref[...]


def _combine(acc, recip_col, h, bias):
    return pl.pallas_call(
        _combine_body,
        out_shape=jax.ShapeDtypeStruct((N, D), jnp.float32),
    )(acc, recip_col, h, bias.reshape(1, D))


@jax.jit
def kernel(x, edge_index, dropout_mask, W, attn_l, attn_r, bias, gamma, beta):
    h, feat, el, er = _dense(x, dropout_mask, W, attn_l, attn_r, gamma, beta)
    el_pad = jnp.pad(el.reshape(N), (0, NDEN - N))
    er_pad = jnp.pad(er.reshape(N), (0, NDEN - N))
    src = jnp.pad(edge_index[0].astype(jnp.int32).reshape(NW, EPW),
                  ((0, 0), (0, EPW_PAD - EPW))).reshape(NW, NBLKS, BLK, CHUNK)
    dst = jnp.pad(edge_index[1].astype(jnp.int32).reshape(NW, EPW),
                  ((0, 0), (0, EPW_PAD - EPW)),
                  constant_values=N).reshape(NW, NBLKS, BLK, CHUNK)
    acc, den_all = _sc_edges(feat, el_pad, er_pad, src, dst)
    recip_col = _recip(den_all)[0, :N].reshape(N, 1)
    return _combine(acc, recip_col, h, bias)


# speedup vs baseline: 1.5430x; 1.5430x over previous
import dataclasses

import jax
import jax.numpy as jnp
from jax import lax
from jax.experimental import pallas as pl
from jax.experimental.pallas import tpu as pltpu
from jax.experimental.pallas import tpu_sc as plsc

N = 10000
D = 128
E = 320000
EPS = 1e-5
NEG_SLOPE = 0.2

NC = 2
NS = 16
L = 16
NW = NC * NS
EPW = E // NW
CHUNK = 80
NCHUNK = EPW // CHUNK
BLK = 25
NBLK = NCHUNK // BLK
GROUPS = CHUNK // L
STRIPE = 632
N_PAD = NS * STRIPE


def _dense_body(x_ref, mask_ref, w_ref, al_ref, ar_ref, gamma_ref, beta_ref,
                h_ref, feat_ref, el_ref, er_ref):
    x = x_ref[...]
    mean = jnp.mean(x, axis=0, keepdims=True)
    var = jnp.mean((x - mean) ** 2, axis=0, keepdims=True)
    h = (x - mean) * lax.rsqrt(var + EPS) * gamma_ref[...] + beta_ref[...]
    h = jnp.maximum(h, 0.0) * mask_ref[...]
    h_ref[...] = h
    feat = jnp.dot(h, w_ref[...], preferred_element_type=jnp.float32)
    feat_ref[...] = feat
    el_ref[...] = jnp.sum(feat * al_ref[...], axis=1, keepdims=True)
    er_ref[...] = jnp.sum(feat * ar_ref[...], axis=1, keepdims=True)


def _dense(x, dropout_mask, W, attn_l, attn_r, gamma, beta):
    return pl.pallas_call(
        _dense_body,
        out_shape=[
            jax.ShapeDtypeStruct((N, D), jnp.float32),
            jax.ShapeDtypeStruct((N, D), jnp.float32),
            jax.ShapeDtypeStruct((N, 1), jnp.float32),
            jax.ShapeDtypeStruct((N, 1), jnp.float32),
        ],
    )(x, dropout_mask, W, attn_l.reshape(1, D), attn_r.reshape(1, D),
      gamma.reshape(1, D), beta.reshape(1, D))


def _sc_body(feat_hbm, el_hbm, er_hbm, src_hbm, dst_hbm, zeros_hbm,
             out_hbm, den_hbm,
             el_v, er_v, src_v, dst_v, rows_v, den_v, acc_sh, sem):
    c = lax.axis_index("c")
    s = lax.axis_index("s")
    w = c * NS + s

    pltpu.sync_copy(el_hbm, el_v)
    pltpu.sync_copy(er_hbm, er_v)

    zv = jnp.zeros((L,), jnp.float32)

    @pl.loop(0, N, step=L)
    def _zero(i):
        den_v[pl.ds(i, L)] = zv

    r0 = s * STRIPE
    pltpu.sync_copy(zeros_hbm.at[pl.ds(r0, STRIPE)],
                    acc_sh.at[pl.ds(r0, STRIPE)])
    plsc.subcore_barrier()

    @pl.loop(0, NBLK)
    def _block(b):
        pltpu.sync_copy(src_hbm.at[w].at[b], src_v)
        pltpu.sync_copy(dst_hbm.at[w].at[b], dst_v)

        @pl.loop(0, BLK)
        def _chunk(ci):
            pltpu.async_copy(feat_hbm.at[src_v.at[ci]], rows_v, sem).wait()
            for g in range(GROUPS):
                src16 = src_v[ci, pl.ds(g * L, L)]
                dst16 = dst_v[ci, pl.ds(g * L, L)]
                e = (plsc.load_gather(el_v, [src16]) +
                     plsc.load_gather(er_v, [dst16]))
                e = jnp.where(e >= 0.0, e, e * NEG_SLOPE)
                wv = jnp.exp(e)
                plsc.addupdate_scatter(den_v, [dst16], wv)
                for j in range(L):
                    r = g * L + j
                    wj = wv[j]
                    for k in range(D // L):
                        sl = pl.ds(k * L, L)
                        rows_v[r, sl] = rows_v[r, sl] * wj
            pltpu.sync_copy(rows_v, acc_sh.at[dst_v.at[ci]], add=True)

    plsc.subcore_barrier()
    pltpu.sync_copy(acc_sh.at[pl.ds(r0, STRIPE)],
                    out_hbm.at[c, pl.ds(r0, STRIPE)])
    pltpu.sync_copy(den_v, den_hbm.at[w])


def _sc_edges(feat, el, er, src, dst, zeros):
    mesh = plsc.VectorSubcoreMesh(core_axis_name="c", subcore_axis_name="s",
                                  num_cores=NC)
    cp = pltpu.CompilerParams()
    if "needs_layout_passes" in pltpu.CompilerParams.__dataclass_fields__:
        cp = dataclasses.replace(cp, needs_layout_passes=False)
    kern = pl.kernel(
        _sc_body,
        out_type=[
            jax.ShapeDtypeStruct((NC, N_PAD, D), jnp.float32),
            jax.ShapeDtypeStruct((NW, N), jnp.float32),
        ],
        mesh=mesh,
        compiler_params=cp,
        scratch_types=[
            pltpu.VMEM((N,), jnp.float32),
            pltpu.VMEM((N,), jnp.float32),
            pltpu.VMEM((BLK, CHUNK), jnp.int32),
            pltpu.VMEM((BLK, CHUNK), jnp.int32),
            pltpu.VMEM((CHUNK, D), jnp.float32),
            pltpu.VMEM((N,), jnp.float32),
            pltpu.VMEM_SHARED((N_PAD, D), jnp.float32),
            pltpu.SemaphoreType.DMA,
        ],
    )
    return kern(feat, el, er, src, dst, zeros)


def _recip_body(den_ref, out_ref):
    den = jnp.sum(den_ref[...], axis=0, keepdims=True)
    out_ref[...] = jnp.where(den == 0.0, 0.0, 1.0 / den)


def _recip(den_all):
    return pl.pallas_call(
        _recip_body,
        out_shape=jax.ShapeDtypeStruct((1, N), jnp.float32),
    )(den_all)


def _combine_body(n0_ref, n1_ref, recip_ref, h_ref, bias_ref, out_ref):
    num = n0_ref[...] + n1_ref[...]
    out_ref[...] = num * recip_ref[...] + h_ref[...] + bias_ref[...]


def _combine(n0, n1, recip_col, h, bias):
    return pl.pallas_call(
        _combine_body,
        out_shape=jax.ShapeDtypeStruct((N, D), jnp.float32),
    )(n0, n1, recip_col, h, bias.reshape(1, D))


@jax.jit
def kernel(x, edge_index, dropout_mask, W, attn_l, attn_r, bias, gamma, beta):
    h, feat, el, er = _dense(x, dropout_mask, W, attn_l, attn_r, gamma, beta)
    src = edge_index[0].astype(jnp.int32).reshape(NW, NBLK, BLK, CHUNK)
    dst = edge_index[1].astype(jnp.int32).reshape(NW, NBLK, BLK, CHUNK)
    zeros = jnp.zeros((N_PAD, D), jnp.float32)
    acc, den_all = _sc_edges(feat, el.reshape(N), er.reshape(N), src, dst, zeros)
    recip_col = _recip(den_all).reshape(N, 1)
    return _combine(acc[0, :N], acc[1, :N], recip_col, h, bias)


# speedup vs baseline: 1.6209x; 1.0505x over previous
import dataclasses

import jax
import jax.numpy as jnp
from jax import lax
from jax.experimental import pallas as pl
from jax.experimental.pallas import tpu as pltpu
from jax.experimental.pallas import tpu_sc as plsc

N = 10000
D = 128
E = 320000
EPS = 1e-5
NEG_SLOPE = 0.2

NC = 2
NS = 16
L = 16
NW = NC * NS
EPW = E // NW
CHUNK = 80
NCHUNK = EPW // CHUNK
BLK = 25
NBLK = NCHUNK // BLK
GROUPS = CHUNK // L
STRIPE = 632
N_PAD = NS * STRIPE


def _dense_body(x_ref, mask_ref, w_ref, al_ref, ar_ref, gamma_ref, beta_ref,
                h_ref, feat_ref, el_ref, er_ref):
    x = x_ref[...]
    mean = jnp.mean(x, axis=0, keepdims=True)
    var = jnp.mean((x - mean) ** 2, axis=0, keepdims=True)
    h = (x - mean) * lax.rsqrt(var + EPS) * gamma_ref[...] + beta_ref[...]
    h = jnp.maximum(h, 0.0) * mask_ref[...]
    h_ref[...] = h
    feat = jnp.dot(h, w_ref[...], preferred_element_type=jnp.float32)
    feat_ref[...] = feat
    el_ref[...] = jnp.sum(feat * al_ref[...], axis=1, keepdims=True)
    er_ref[...] = jnp.sum(feat * ar_ref[...], axis=1, keepdims=True)


def _dense(x, dropout_mask, W, attn_l, attn_r, gamma, beta):
    return pl.pallas_call(
        _dense_body,
        out_shape=[
            jax.ShapeDtypeStruct((N, D), jnp.float32),
            jax.ShapeDtypeStruct((N, D), jnp.float32),
            jax.ShapeDtypeStruct((N, 1), jnp.float32),
            jax.ShapeDtypeStruct((N, 1), jnp.float32),
        ],
    )(x, dropout_mask, W, attn_l.reshape(1, D), attn_r.reshape(1, D),
      gamma.reshape(1, D), beta.reshape(1, D))


def _sc_body(feat_hbm, el_hbm, er_hbm, src_hbm, dst_hbm,
             out_hbm, den_hbm,
             el_v, er_v, src_v, dst_v, rows_v, den_v, acc_sh, sem, sem_e):
    c = lax.axis_index("c")
    s = lax.axis_index("s")
    w = c * NS + s

    pltpu.async_copy(el_hbm, el_v, sem_e)
    pltpu.async_copy(er_hbm, er_v, sem_e)

    zv = jnp.zeros((L,), jnp.float32)

    @pl.loop(0, N, step=L)
    def _zero(i):
        den_v[pl.ds(i, L)] = zv

    @pl.loop(0, CHUNK)
    def _zero_rows(r):
        @pl.loop(0, D, step=L)
        def _zr(k):
            rows_v[r, pl.ds(k, L)] = zv

    r0 = s * STRIPE

    @pl.loop(0, (STRIPE // CHUNK) * CHUNK, step=CHUNK)
    def _zero_acc(i):
        pltpu.sync_copy(rows_v, acc_sh.at[pl.ds(r0 + i, CHUNK)])

    _REM = STRIPE - (STRIPE // CHUNK) * CHUNK
    pltpu.sync_copy(rows_v.at[pl.ds(0, _REM)],
                    acc_sh.at[pl.ds(r0 + STRIPE - _REM, _REM)])
    pltpu.make_async_copy(el_hbm, el_v, sem_e).wait()
    pltpu.make_async_copy(er_hbm, er_v, sem_e).wait()
    plsc.subcore_barrier()

    @pl.loop(0, NBLK)
    def _block(b):
        pltpu.sync_copy(src_hbm.at[w].at[b], src_v)
        pltpu.sync_copy(dst_hbm.at[w].at[b], dst_v)

        @pl.loop(0, BLK)
        def _chunk(ci):
            pltpu.async_copy(feat_hbm.at[src_v.at[ci]], rows_v, sem).wait()
            for g in range(GROUPS):
                src16 = src_v[ci, pl.ds(g * L, L)]
                dst16 = dst_v[ci, pl.ds(g * L, L)]
                e = (plsc.load_gather(el_v, [src16]) +
                     plsc.load_gather(er_v, [dst16]))
                e = jnp.where(e >= 0.0, e, e * NEG_SLOPE)
                wv = jnp.exp(e)
                plsc.addupdate_scatter(den_v, [dst16], wv)
                for j in range(L):
                    r = g * L + j
                    wj = wv[j]
                    for k in range(D // L):
                        sl = pl.ds(k * L, L)
                        rows_v[r, sl] = rows_v[r, sl] * wj
            pltpu.sync_copy(rows_v, acc_sh.at[dst_v.at[ci]], add=True)

    plsc.subcore_barrier()
    pltpu.sync_copy(acc_sh.at[pl.ds(r0, STRIPE)],
                    out_hbm.at[c, pl.ds(r0, STRIPE)])
    pltpu.sync_copy(den_v, den_hbm.at[w])


def _sc_edges(feat, el, er, src, dst):
    mesh = plsc.VectorSubcoreMesh(core_axis_name="c", subcore_axis_name="s",
                                  num_cores=NC)
    cp = pltpu.CompilerParams()
    if "needs_layout_passes" in pltpu.CompilerParams.__dataclass_fields__:
        cp = dataclasses.replace(cp, needs_layout_passes=False)
    kern = pl.kernel(
        _sc_body,
        out_type=[
            jax.ShapeDtypeStruct((NC, N_PAD, D), jnp.float32),
            jax.ShapeDtypeStruct((NW, N), jnp.float32),
        ],
        mesh=mesh,
        compiler_params=cp,
        scratch_types=[
            pltpu.VMEM((N,), jnp.float32),
            pltpu.VMEM((N,), jnp.float32),
            pltpu.VMEM((BLK, CHUNK), jnp.int32),
            pltpu.VMEM((BLK, CHUNK), jnp.int32),
            pltpu.VMEM((CHUNK, D), jnp.float32),
            pltpu.VMEM((N,), jnp.float32),
            pltpu.VMEM_SHARED((N_PAD, D), jnp.float32),
            pltpu.SemaphoreType.DMA,
            pltpu.SemaphoreType.DMA,
        ],
    )
    return kern(feat, el, er, src, dst)


def _recip_body(den_ref, out_ref):
    den = jnp.sum(den_ref[...], axis=0, keepdims=True)
    out_ref[...] = jnp.where(den == 0.0, 0.0, 1.0 / den)


def _recip(den_all):
    return pl.pallas_call(
        _recip_body,
        out_shape=jax.ShapeDtypeStruct((1, N), jnp.float32),
    )(den_all)


def _combine_body(acc_ref, recip_ref, h_ref, bias_ref, out_ref):
    num = acc_ref[0, :N, :] + acc_ref[1, :N, :]
    out_ref[...] = num * recip_ref[...] + h_ref[...] + bias_ref[...]


def _combine(acc, recip_col, h, bias):
    return pl.pallas_call(
        _combine_body,
        out_shape=jax.ShapeDtypeStruct((N, D), jnp.float32),
    )(acc, recip_col, h, bias.reshape(1, D))


@jax.jit
def kernel(x, edge_index, dropout_mask, W, attn_l, attn_r, bias, gamma, beta):
    h, feat, el, er = _dense(x, dropout_mask, W, attn_l, attn_r, gamma, beta)
    src = edge_index[0].astype(jnp.int32).reshape(NW, NBLK, BLK, CHUNK)
    dst = edge_index[1].astype(jnp.int32).reshape(NW, NBLK, BLK, CHUNK)
    acc, den_all = _sc_edges(feat, el.reshape(N), er.reshape(N), src, dst)
    recip_col = _recip(den_all).reshape(N, 1)
    return _combine(acc, recip_col, h, bias)


# speedup vs baseline: 1.6901x; 1.0427x over previous
import dataclasses

import jax
import jax.numpy as jnp
from jax import lax
from jax.experimental import pallas as pl
from jax.experimental.pallas import tpu as pltpu
from jax.experimental.pallas import tpu_sc as plsc

N = 10000
D = 128
E = 320000
EPS = 1e-5
NEG_SLOPE = 0.2

NC = 2
NS = 16
L = 16
NW = NC * NS
EPW = E // NW
CHUNK = 80
NCHUNK = EPW // CHUNK
BLK = 25
NBLK = NCHUNK // BLK
GROUPS = CHUNK // L
STRIPE = 632
N_PAD = NS * STRIPE


def _dense_body(x_ref, mask_ref, w_ref, al_ref, ar_ref, gamma_ref, beta_ref,
                h_ref, feat_ref, el_ref, er_ref):
    x = x_ref[...]
    mean = jnp.mean(x, axis=0, keepdims=True)
    var = jnp.mean((x - mean) ** 2, axis=0, keepdims=True)
    h = (x - mean) * lax.rsqrt(var + EPS) * gamma_ref[...] + beta_ref[...]
    h = jnp.maximum(h, 0.0) * mask_ref[...]
    h_ref[...] = h
    feat = jnp.dot(h, w_ref[...], preferred_element_type=jnp.float32)
    feat_ref[...] = feat
    el_ref[...] = jnp.sum(feat * al_ref[...], axis=1, keepdims=True)
    er_ref[...] = jnp.sum(feat * ar_ref[...], axis=1, keepdims=True)


def _dense(x, dropout_mask, W, attn_l, attn_r, gamma, beta):
    return pl.pallas_call(
        _dense_body,
        out_shape=[
            jax.ShapeDtypeStruct((N, D), jnp.float32),
            jax.ShapeDtypeStruct((N, D), jnp.float32),
            jax.ShapeDtypeStruct((N, 1), jnp.float32),
            jax.ShapeDtypeStruct((N, 1), jnp.float32),
        ],
    )(x, dropout_mask, W, attn_l.reshape(1, D), attn_r.reshape(1, D),
      gamma.reshape(1, D), beta.reshape(1, D))


def _sc_body(feat_hbm, el_hbm, er_hbm, src_hbm, dst_hbm,
             out_hbm, den_hbm,
             el_v, er_v, src_v, dst_v, rows_v, wbuf_v, den_v, acc_sh,
             sem, sem_e):
    c = lax.axis_index("c")
    s = lax.axis_index("s")
    w = c * NS + s

    pltpu.async_copy(el_hbm, el_v, sem_e)
    pltpu.async_copy(er_hbm, er_v, sem_e)

    zv = jnp.zeros((L,), jnp.float32)

    @pl.loop(0, N, step=L)
    def _zero(i):
        den_v[pl.ds(i, L)] = zv

    @pl.loop(0, CHUNK)
    def _zero_rows(r):
        @pl.loop(0, D, step=L)
        def _zr(k):
            rows_v[r, pl.ds(k, L)] = zv

    r0 = s * STRIPE

    @pl.loop(0, (STRIPE // CHUNK) * CHUNK, step=CHUNK)
    def _zero_acc(i):
        pltpu.sync_copy(rows_v, acc_sh.at[pl.ds(r0 + i, CHUNK)])

    _REM = STRIPE - (STRIPE // CHUNK) * CHUNK
    pltpu.sync_copy(rows_v.at[pl.ds(0, _REM)],
                    acc_sh.at[pl.ds(r0 + STRIPE - _REM, _REM)])
    pltpu.make_async_copy(el_hbm, el_v, sem_e).wait()
    pltpu.make_async_copy(er_hbm, er_v, sem_e).wait()
    plsc.subcore_barrier()

    @pl.loop(0, NBLK)
    def _block(b):
        pltpu.sync_copy(src_hbm.at[w].at[b], src_v)
        pltpu.sync_copy(dst_hbm.at[w].at[b], dst_v)

        @pl.loop(0, BLK)
        def _chunk(ci):
            pltpu.async_copy(feat_hbm.at[src_v.at[ci]], rows_v, sem)
            for g in range(GROUPS):
                src16 = src_v[ci, pl.ds(g * L, L)]
                dst16 = dst_v[ci, pl.ds(g * L, L)]
                e = (plsc.load_gather(el_v, [src16]) +
                     plsc.load_gather(er_v, [dst16]))
                e = jnp.where(e >= 0.0, e, e * NEG_SLOPE)
                wv = jnp.exp(e)
                plsc.addupdate_scatter(den_v, [dst16], wv)
                wbuf_v[pl.ds(g * L, L)] = wv
            pltpu.make_async_copy(feat_hbm.at[src_v.at[ci]], rows_v,
                                  sem).wait()
            for g in range(GROUPS):
                wg = wbuf_v[pl.ds(g * L, L)]
                for j in range(L):
                    r = g * L + j
                    wj = wg[j]
                    for k in range(D // L):
                        sl = pl.ds(k * L, L)
                        rows_v[r, sl] = rows_v[r, sl] * wj
            pltpu.sync_copy(rows_v, acc_sh.at[dst_v.at[ci]], add=True)

    plsc.subcore_barrier()
    pltpu.sync_copy(acc_sh.at[pl.ds(r0, STRIPE)],
                    out_hbm.at[c, pl.ds(r0, STRIPE)])
    pltpu.sync_copy(den_v, den_hbm.at[w])


def _sc_edges(feat, el, er, src, dst):
    mesh = plsc.VectorSubcoreMesh(core_axis_name="c", subcore_axis_name="s",
                                  num_cores=NC)
    cp = pltpu.CompilerParams()
    if "needs_layout_passes" in pltpu.CompilerParams.__dataclass_fields__:
        cp = dataclasses.replace(cp, needs_layout_passes=False)
    kern = pl.kernel(
        _sc_body,
        out_type=[
            jax.ShapeDtypeStruct((NC, N_PAD, D), jnp.float32),
            jax.ShapeDtypeStruct((NW, N), jnp.float32),
        ],
        mesh=mesh,
        compiler_params=cp,
        scratch_types=[
            pltpu.VMEM((N,), jnp.float32),
            pltpu.VMEM((N,), jnp.float32),
            pltpu.VMEM((BLK, CHUNK), jnp.int32),
            pltpu.VMEM((BLK, CHUNK), jnp.int32),
            pltpu.VMEM((CHUNK, D), jnp.float32),
            pltpu.VMEM((CHUNK,), jnp.float32),
            pltpu.VMEM((N,), jnp.float32),
            pltpu.VMEM_SHARED((N_PAD, D), jnp.float32),
            pltpu.SemaphoreType.DMA,
            pltpu.SemaphoreType.DMA,
        ],
    )
    return kern(feat, el, er, src, dst)


def _recip_body(den_ref, out_ref):
    den = jnp.sum(den_ref[...], axis=0, keepdims=True)
    out_ref[...] = jnp.where(den == 0.0, 0.0, 1.0 / den)


def _recip(den_all):
    return pl.pallas_call(
        _recip_body,
        out_shape=jax.ShapeDtypeStruct((1, N), jnp.float32),
    )(den_all)


def _combine_body(acc_ref, recip_ref, h_ref, bias_ref, out_ref):
    num = acc_ref[0, :N, :] + acc_ref[1, :N, :]
    out_ref[...] = num * recip_ref[...] + h_ref[...] + bias_ref[...]


def _combine(acc, recip_col, h, bias):
    return pl.pallas_call(
        _combine_body,
        out_shape=jax.ShapeDtypeStruct((N, D), jnp.float32),
    )(acc, recip_col, h, bias.reshape(1, D))


@jax.jit
def kernel(x, edge_index, dropout_mask, W, attn_l, attn_r, bias, gamma, beta):
    h, feat, el, er = _dense(x, dropout_mask, W, attn_l, attn_r, gamma, beta)
    src = edge_index[0].astype(jnp.int32).reshape(NW, NBLK, BLK, CHUNK)
    dst = edge_index[1].astype(jnp.int32).reshape(NW, NBLK, BLK, CHUNK)
    acc, den_all = _sc_edges(feat, el.reshape(N), er.reshape(N), src, dst)
    recip_col = _recip(den_all).reshape(N, 1)
    return _combine(acc, recip_col, h, bias)
